# baseline (speedup 1.0000x reference)
import functools

import jax
import jax.numpy as jnp
from jax import lax
from jax.experimental import pallas as pl
from jax.experimental.pallas import tpu as pltpu

BIN_SIZE = 20
MAX_DURATION = 1000
NB_BINS = MAX_DURATION // BIN_SIZE + 1

NBLK = 128
BB = 32


def _bin_kernel(bin_ref, spikes_ref, out_ref, s_ref, *, ng):
    w = NBLK * NB_BINS

    @pl.when(pl.program_id(0) == 0)
    def _build_scatter_matrix():
        n = lax.broadcasted_iota(jnp.int32, (NBLK, w), 0)
        k = lax.broadcasted_iota(jnp.int32, (NBLK, w), 1)
        s_ref[...] = (k == n * NB_BINS + bin_ref[0]).astype(jnp.float32)

    for g in range(ng):
        out_ref[:, g * w:(g + 1) * w] = jnp.dot(
            spikes_ref[:, g * NBLK:(g + 1) * NBLK],
            s_ref[...],
            preferred_element_type=jnp.float32,
        )


def kernel(spikes, time):
    B, N = spikes.shape
    bin_idx = (jnp.asarray(time, jnp.int32) // BIN_SIZE).reshape((1,))

    ng = N // NBLK
    grid = (B // BB,)

    out = pl.pallas_call(
        functools.partial(_bin_kernel, ng=ng),
        grid=grid,
        in_specs=[
            pl.BlockSpec(memory_space=pltpu.SMEM),
            pl.BlockSpec((BB, N), lambda i: (i, 0)),
        ],
        out_specs=pl.BlockSpec((BB, N * NB_BINS), lambda i: (i, 0)),
        out_shape=jax.ShapeDtypeStruct((B, N * NB_BINS), spikes.dtype),
        scratch_shapes=[pltpu.VMEM((NBLK, NBLK * NB_BINS), jnp.float32)],
    )(bin_idx, spikes)
    return out


# speedup vs baseline: 1.0330x; 1.0330x over previous
import functools

import jax
import jax.numpy as jnp
from jax import lax
from jax.experimental import pallas as pl
from jax.experimental.pallas import tpu as pltpu

BIN_SIZE = 20
MAX_DURATION = 1000
NB_BINS = MAX_DURATION // BIN_SIZE + 1

NBLK = 128
BB = 128


def _bin_kernel(bin_ref, spikes_ref, out_ref, s_ref, *, ng):
    w = NBLK * NB_BINS

    @pl.when(pl.program_id(0) == 0)
    def _build_scatter_matrix():
        n = lax.broadcasted_iota(jnp.int32, (NBLK, w), 0)
        k = lax.broadcasted_iota(jnp.int32, (NBLK, w), 1)
        s_ref[...] = (k == n * NB_BINS + bin_ref[0]).astype(jnp.float32)

    for g in range(ng):
        out_ref[:, g * w:(g + 1) * w] = jnp.dot(
            spikes_ref[:, g * NBLK:(g + 1) * NBLK],
            s_ref[...],
            preferred_element_type=jnp.float32,
        )


def kernel(spikes, time):
    B, N = spikes.shape
    bin_idx = (jnp.asarray(time, jnp.int32) // BIN_SIZE).reshape((1,))

    ng = N // NBLK
    grid = (B // BB,)

    out = pl.pallas_call(
        functools.partial(_bin_kernel, ng=ng),
        grid=grid,
        in_specs=[
            pl.BlockSpec(memory_space=pltpu.SMEM),
            pl.BlockSpec((BB, N), lambda i: (i, 0)),
        ],
        out_specs=pl.BlockSpec((BB, N * NB_BINS), lambda i: (i, 0)),
        out_shape=jax.ShapeDtypeStruct((B, N * NB_BINS), spikes.dtype),
        scratch_shapes=[pltpu.VMEM((NBLK, NBLK * NB_BINS), jnp.float32)],
    )(bin_idx, spikes)
    return out


# speedup vs baseline: 1.1032x; 1.0680x over previous
import functools

import jax
import jax.numpy as jnp
from jax import lax
from jax.experimental import pallas as pl
from jax.experimental.pallas import tpu as pltpu

BIN_SIZE = 20
MAX_DURATION = 1000
NB_BINS = MAX_DURATION // BIN_SIZE + 1

NBLK = 128
BB = 64


def _bin_kernel(bin_ref, spikes_ref, out_ref, s_ref, *, ng):
    w = NBLK * NB_BINS

    @pl.when(pl.program_id(0) == 0)
    def _build_scatter_matrix():
        n = lax.broadcasted_iota(jnp.int32, (NBLK, w), 0)
        k = lax.broadcasted_iota(jnp.int32, (NBLK, w), 1)
        s_ref[...] = (k == n * NB_BINS + bin_ref[0]).astype(jnp.float32)

    for g in range(ng):
        out_ref[:, g * w:(g + 1) * w] = jnp.dot(
            spikes_ref[:, g * NBLK:(g + 1) * NBLK],
            s_ref[...],
            preferred_element_type=jnp.float32,
        )


def kernel(spikes, time):
    B, N = spikes.shape
    bin_idx = (jnp.asarray(time, jnp.int32) // BIN_SIZE).reshape((1,))

    ng = N // NBLK
    grid = (B // BB,)

    out = pl.pallas_call(
        functools.partial(_bin_kernel, ng=ng),
        grid=grid,
        in_specs=[
            pl.BlockSpec(memory_space=pltpu.SMEM),
            pl.BlockSpec((BB, N), lambda i: (i, 0)),
        ],
        out_specs=pl.BlockSpec((BB, N * NB_BINS), lambda i: (i, 0)),
        out_shape=jax.ShapeDtypeStruct((B, N * NB_BINS), spikes.dtype),
        scratch_shapes=[pltpu.VMEM((NBLK, NBLK * NB_BINS), jnp.float32)],
    )(bin_idx, spikes)
    return out
